# baseline (speedup 1.0000x reference)
import functools

import jax
import jax.numpy as jnp
from jax import lax
from jax.experimental import pallas as pl
from jax.experimental.pallas import tpu as pltpu
from jax.experimental.pallas import tpu_sc as plsc

N_NODES = 10000
N_EDGES = 160000
D = 256
HALF = 128
N_LAYERS = 4
N_GRAPHS = 64
EPS = 1e-5

NPAD = 10240
E2 = N_EDGES + N_NODES
N_TILES = 16

ACH = 96
NCH = 112
ETOT = N_TILES * NCH * ACH
assert ETOT == 172032
NSTRIP = 8
ROWS_PER_TILE = NPAD // N_TILES
ZB = 64
NZB = ROWS_PER_TILE // ZB

PCH = 128
PCHUNKS = ETOT // (32 * PCH)

_MESH = plsc.VectorSubcoreMesh(core_axis_name="c", subcore_axis_name="s")
_SC_PARAMS = pltpu.CompilerParams(needs_layout_passes=False)


@functools.partial(
    pl.kernel,
    out_type=jax.ShapeDtypeStruct((32, NPAD), jnp.float32),
    mesh=_MESH,
    compiler_params=_SC_PARAMS,
    scratch_types=[
        pltpu.VMEM((PCHUNKS, PCH), jnp.int32),
        pltpu.VMEM((PCHUNKS, PCH), jnp.float32),
        pltpu.VMEM((NPAD,), jnp.float32),
    ],
)
def _deg_kernel(dst_hbm, ew_hbm, out_hbm, dstv, eww, degv):
    c = lax.axis_index("c")
    s = lax.axis_index("s")
    w = s * 2 + c

    def zero_body(i, _):
        degv[pl.ds(i * 16, 16)] = jnp.zeros((16,), jnp.float32)
        return 0

    lax.fori_loop(0, NPAD // 16, zero_body, 0)

    pltpu.sync_copy(dst_hbm.at[w], dstv)
    pltpu.sync_copy(ew_hbm.at[w], eww)

    def body(k, _):
        for u in range(PCH // 16):
            d16 = dstv[k, pl.ds(u * 16, 16)]
            e16 = eww[k, pl.ds(u * 16, 16)]
            plsc.addupdate_scatter(degv, [d16], e16)
        return 0

    lax.fori_loop(0, PCHUNKS, body, 0)
    pltpu.sync_copy(degv, out_hbm.at[w])


def _finalize_deg_body(part_ref, dinv_ref):
    deg = jnp.sum(part_ref[...], axis=0, keepdims=True)
    dinv_ref[...] = jnp.where(deg > 0, 1.0 / jnp.sqrt(deg), 0.0)


def _finalize_deg(partials):
    return pl.pallas_call(
        _finalize_deg_body,
        out_shape=jax.ShapeDtypeStruct((1, NPAD), jnp.float32),
    )(partials)


@functools.partial(
    pl.kernel,
    out_type=jax.ShapeDtypeStruct((32, PCHUNKS, PCH), jnp.float32),
    mesh=_MESH,
    compiler_params=_SC_PARAMS,
    scratch_types=[
        pltpu.VMEM((PCHUNKS, PCH), jnp.int32),
        pltpu.VMEM((PCHUNKS, PCH), jnp.int32),
        pltpu.VMEM((PCHUNKS, PCH), jnp.float32),
        pltpu.VMEM((PCHUNKS, PCH), jnp.float32),
        pltpu.VMEM((NPAD,), jnp.float32),
    ],
)
def _norm_kernel(src_hbm, dst_hbm, ew_hbm, dinv_hbm, out_hbm, srcv, dstv, eww, normv, dinvv):
    c = lax.axis_index("c")
    s = lax.axis_index("s")
    w = s * 2 + c

    pltpu.sync_copy(dinv_hbm, dinvv)
    pltpu.sync_copy(src_hbm.at[w], srcv)
    pltpu.sync_copy(dst_hbm.at[w], dstv)
    pltpu.sync_copy(ew_hbm.at[w], eww)

    def body(k, _):
        for u in range(PCH // 16):
            sl = pl.ds(u * 16, 16)
            s16 = srcv[k, sl]
            d16 = dstv[k, sl]
            e16 = eww[k, sl]
            nv = plsc.load_gather(dinvv, [s16]) * e16 * plsc.load_gather(dinvv, [d16])
            normv[k, sl] = nv
        return 0

    lax.fori_loop(0, PCHUNKS, body, 0)
    pltpu.sync_copy(normv, out_hbm.at[w])


@functools.partial(
    pl.kernel,
    out_type=jax.ShapeDtypeStruct((2, NPAD, HALF), jnp.float32),
    mesh=_MESH,
    compiler_params=_SC_PARAMS,
    scratch_types=[
        pltpu.VMEM((NSTRIP, ACH), jnp.int32),
        pltpu.VMEM((NSTRIP, ACH), jnp.int32),
        pltpu.VMEM((NSTRIP, ACH), jnp.float32),
        pltpu.VMEM((ACH, HALF), jnp.float32),
        pltpu.VMEM((ACH, HALF), jnp.float32),
        pltpu.VMEM_SHARED((NPAD, HALF), jnp.float32),
        pltpu.SemaphoreType.DMA,
        pltpu.SemaphoreType.DMA,
        pltpu.SemaphoreType.DMA,
        pltpu.SemaphoreType.DMA,
    ],
)
def _agg_kernel(h_hbm, src_hbm, dst_hbm, norm_hbm, out_hbm,
                srcv, dstv, normv, gb0, gb1, acc, gs0, gs1, ss0, ss1):
    c = lax.axis_index("c")
    s = lax.axis_index("s")
    gbufs = (gb0, gb1)
    gsems = (gs0, gs1)
    ssems = (ss0, ss1)

    def zbody(r, _):
        for u in range(HALF // 16):
            gb0[r, pl.ds(u * 16, 16)] = jnp.zeros((16,), jnp.float32)
        return 0

    lax.fori_loop(0, ACH, zbody, 0)
    for q in range(NZB):
        pltpu.sync_copy(gb0.at[pl.ds(0, ZB)], acc.at[pl.ds(s * ROWS_PER_TILE + q * ZB, ZB)])
    plsc.subcore_barrier()

    def _gather(k2, b):
        pltpu.async_copy(h_hbm.at[c].at[srcv.at[k2]], gbufs[b], gsems[b])

    def _wait_gather(k2, b):
        pltpu.make_async_copy(h_hbm.at[c].at[srcv.at[k2]], gbufs[b], gsems[b]).wait()

    def _scatter(k2, b):
        pltpu.async_copy(gbufs[b], acc.at[dstv.at[k2]], ssems[b], add=True)

    def _wait_scatter(b):
        pltpu.make_async_copy(gbufs[b], acc.at[dstv.at[0]], ssems[b]).wait()

    def strip_body(st, _):
        off = pl.ds(pl.multiple_of(st * NSTRIP, NSTRIP), NSTRIP)
        pltpu.sync_copy(src_hbm.at[s].at[off], srcv)
        pltpu.sync_copy(dst_hbm.at[s].at[off], dstv)
        pltpu.sync_copy(norm_hbm.at[s].at[off], normv)
        _gather(0, 0)
        for k2 in range(NSTRIP):
            b = k2 % 2
            if k2 < NSTRIP - 1:
                if k2 >= 1:
                    _wait_scatter(1 - b)
                _gather(k2 + 1, 1 - b)
            _wait_gather(k2, b)

            idxk = jnp.zeros((16,), jnp.int32) + k2

            def rowbody(r, _, b=b, idxk=idxk):
                sc16 = plsc.load_gather(normv, [idxk, jnp.zeros((16,), jnp.int32) + r])
                for u in range(HALF // 16):
                    sl = pl.ds(u * 16, 16)
                    gbufs[b][r, sl] = gbufs[b][r, sl] * sc16
                return 0

            lax.fori_loop(0, ACH, rowbody, 0)

            _scatter(k2, b)
        _wait_scatter(0)
        _wait_scatter(1)
        return 0

    lax.fori_loop(0, NCH // NSTRIP, strip_body, 0)
    plsc.subcore_barrier()

    for q in range(NZB):
        rows = pl.ds(s * ROWS_PER_TILE + q * ZB, ZB)
        pltpu.sync_copy(acc.at[rows], gb0.at[pl.ds(0, ZB)])
        pltpu.sync_copy(gb0.at[pl.ds(0, ZB)], out_hbm.at[c].at[rows])


_GRID = NPAD // 1024
_NB = 1024


def _matmul_body(x_ref, w_ref, h_ref):
    x0 = x_ref[0]
    x1 = x_ref[1]
    h_ref[0, :, :] = (
        jnp.dot(x0, w_ref[0, 0], preferred_element_type=jnp.float32)
        + jnp.dot(x1, w_ref[1, 0], preferred_element_type=jnp.float32)
    )
    h_ref[1, :, :] = (
        jnp.dot(x0, w_ref[0, 1], preferred_element_type=jnp.float32)
        + jnp.dot(x1, w_ref[1, 1], preferred_element_type=jnp.float32)
    )


def _matmul(x, wq):
    return pl.pallas_call(
        _matmul_body,
        grid=(_GRID,),
        in_specs=[
            pl.BlockSpec((2, _NB, HALF), lambda i: (0, i, 0)),
            pl.BlockSpec((2, 2, HALF, HALF), lambda i: (0, 0, 0, 0)),
        ],
        out_specs=pl.BlockSpec((2, _NB, HALF), lambda i: (0, i, 0)),
        out_shape=jax.ShapeDtypeStruct((2, NPAD, HALF), jnp.float32),
    )(x, wq)


def _gn_stats_body(agg_ref, b_ref, p_ref, y_ref, s1_ref, cnt_ref):
    i = pl.program_id(0)

    @pl.when(i == 0)
    def _():
        s1_ref[...] = jnp.zeros_like(s1_ref)
        cnt_ref[...] = jnp.zeros_like(cnt_ref)

    pb = p_ref[...]
    ones = jnp.ones((_NB, HALF), jnp.float32)
    cnt_ref[...] += jnp.dot(pb, ones, preferred_element_type=jnp.float32, precision=lax.Precision.HIGHEST)
    for f in range(2):
        y = agg_ref[f] + b_ref[f]
        y_ref[f, :, :] = y
        s1_ref[f, :, :] += jnp.dot(pb, y, preferred_element_type=jnp.float32, precision=lax.Precision.HIGHEST)


def _gn_stats(agg, b2, p):
    return pl.pallas_call(
        _gn_stats_body,
        grid=(_GRID,),
        in_specs=[
            pl.BlockSpec((2, _NB, HALF), lambda i: (0, i, 0)),
            pl.BlockSpec((2, 1, HALF), lambda i: (0, 0, 0)),
            pl.BlockSpec((N_GRAPHS, _NB), lambda i: (0, i)),
        ],
        out_specs=[
            pl.BlockSpec((2, _NB, HALF), lambda i: (0, i, 0)),
            pl.BlockSpec((2, N_GRAPHS, HALF), lambda i: (0, 0, 0)),
            pl.BlockSpec((N_GRAPHS, HALF), lambda i: (0, 0)),
        ],
        out_shape=[
            jax.ShapeDtypeStruct((2, NPAD, HALF), jnp.float32),
            jax.ShapeDtypeStruct((2, N_GRAPHS, HALF), jnp.float32),
            jax.ShapeDtypeStruct((N_GRAPHS, HALF), jnp.float32),
        ],
    )(agg, b2, p)


def _gn_var_body(y_ref, p_ref, s1_ref, cnt_ref, gms_ref, s2_ref):
    i = pl.program_id(0)

    @pl.when(i == 0)
    def _():
        s2_ref[...] = jnp.zeros_like(s2_ref)

    pb = p_ref[...]
    cnt = jnp.maximum(cnt_ref[...], 1.0)
    for f in range(2):
        mg = (s1_ref[f] / cnt) * gms_ref[f]
        mb = lax.dot_general(pb, mg, (((0,), (0,)), ((), ())),
                             preferred_element_type=jnp.float32, precision=lax.Precision.HIGHEST)
        o = y_ref[f] - mb
        s2_ref[f, :, :] += jnp.dot(pb, o * o, preferred_element_type=jnp.float32, precision=lax.Precision.HIGHEST)


def _gn_var(y, p, s1, cnt, gms2):
    return pl.pallas_call(
        _gn_var_body,
        grid=(_GRID,),
        in_specs=[
            pl.BlockSpec((2, _NB, HALF), lambda i: (0, i, 0)),
            pl.BlockSpec((N_GRAPHS, _NB), lambda i: (0, i)),
            pl.BlockSpec((2, N_GRAPHS, HALF), lambda i: (0, 0, 0)),
            pl.BlockSpec((N_GRAPHS, HALF), lambda i: (0, 0)),
            pl.BlockSpec((2, 1, HALF), lambda i: (0, 0, 0)),
        ],
        out_specs=pl.BlockSpec((2, N_GRAPHS, HALF), lambda i: (0, 0, 0)),
        out_shape=jax.ShapeDtypeStruct((2, N_GRAPHS, HALF), jnp.float32),
    )(y, p, s1, cnt, gms2)


def _gn_apply_body(y_ref, p_ref, s1_ref, s2_ref, cnt_ref, gms_ref, gw_ref, gb_ref, x_ref):
    pb = p_ref[...]
    cnt = jnp.maximum(cnt_ref[...], 1.0)
    for f in range(2):
        g = gms_ref[f]
        m = s1_ref[f] / cnt
        var = s2_ref[f] / cnt
        rstd = 1.0 / jnp.sqrt(var + EPS)
        mg = m * g
        mb = lax.dot_general(pb, mg, (((0,), (0,)), ((), ())),
                             preferred_element_type=jnp.float32, precision=lax.Precision.HIGHEST)
        rb = lax.dot_general(pb, rstd, (((0,), (0,)), ((), ())),
                             preferred_element_type=jnp.float32, precision=lax.Precision.HIGHEST)
        x = (y_ref[f] - mb) * rb * gw_ref[f] + gb_ref[f]
        x_ref[f, :, :] = jnp.maximum(x, 0.0)


def _gn_apply(y, p, s1, s2, cnt, gms2, gw2, gb2):
    return pl.pallas_call(
        _gn_apply_body,
        grid=(_GRID,),
        in_specs=[
            pl.BlockSpec((2, _NB, HALF), lambda i: (0, i, 0)),
            pl.BlockSpec((N_GRAPHS, _NB), lambda i: (0, i)),
            pl.BlockSpec((2, N_GRAPHS, HALF), lambda i: (0, 0, 0)),
            pl.BlockSpec((2, N_GRAPHS, HALF), lambda i: (0, 0, 0)),
            pl.BlockSpec((N_GRAPHS, HALF), lambda i: (0, 0)),
            pl.BlockSpec((2, 1, HALF), lambda i: (0, 0, 0)),
            pl.BlockSpec((2, 1, HALF), lambda i: (0, 0, 0)),
            pl.BlockSpec((2, 1, HALF), lambda i: (0, 0, 0)),
        ],
        out_specs=pl.BlockSpec((2, _NB, HALF), lambda i: (0, i, 0)),
        out_shape=jax.ShapeDtypeStruct((2, NPAD, HALF), jnp.float32),
    )(y, p, s1, s2, cnt, gms2, gw2, gb2)


def kernel(node, edge_index, edge_attr, batch_ptr, W, b, gn_weight, gn_bias, gn_mean_scale):
    loop = jnp.arange(N_NODES, dtype=jnp.int32)
    src2 = jnp.concatenate([edge_index[0], loop])
    dst2 = jnp.concatenate([edge_index[1], loop])
    ew2 = jnp.concatenate([edge_attr, jnp.ones((N_NODES,), jnp.float32)])
    pad = ETOT - E2
    src_flat = jnp.pad(src2, (0, pad))
    dst_flat = jnp.pad(dst2, (0, pad))
    ew32 = jnp.pad(ew2, (0, pad)).reshape(32, PCHUNKS, PCH)
    src32 = src_flat.reshape(32, PCHUNKS, PCH)
    dst32 = dst_flat.reshape(32, PCHUNKS, PCH)
    src_slab = src_flat.reshape(N_TILES, NCH, ACH)
    dst_slab = dst_flat.reshape(N_TILES, NCH, ACH)

    p = (batch_ptr[None, :] == jnp.arange(N_GRAPHS, dtype=jnp.int32)[:, None])
    p = jnp.pad(p.astype(jnp.float32), ((0, 0), (0, NPAD - N_NODES)))

    xpad = jnp.pad(node, ((0, NPAD - N_NODES), (0, 0)))
    x = jnp.stack([xpad[:, :HALF], xpad[:, HALF:]])

    partials = _deg_kernel(dst32, ew32)
    dinv = _finalize_deg(partials).reshape(NPAD)
    norm_slab = _norm_kernel(src32, dst32, ew32, dinv).reshape(N_TILES, NCH, ACH)

    for l in range(N_LAYERS):
        wq = W[l].reshape(2, HALF, 2, HALF).swapaxes(1, 2)
        b2 = b[l].reshape(2, 1, HALF)
        gms2 = gn_mean_scale[l].reshape(2, 1, HALF)
        gw2 = gn_weight[l].reshape(2, 1, HALF)
        gb2 = gn_bias[l].reshape(2, 1, HALF)

        h = _matmul(x, wq)
        agg = _agg_kernel(h, src_slab, dst_slab, norm_slab)
        y, s1, cnt = _gn_stats(agg, b2, p)
        s2 = _gn_var(y, p, s1, cnt, gms2)
        x = _gn_apply(y, p, s1, s2, cnt, gms2, gw2, gb2)

    return jnp.concatenate([x[0, :N_NODES, :], x[1, :N_NODES, :]], axis=1)


# speedup vs baseline: 1.0056x; 1.0056x over previous
import functools

import jax
import jax.numpy as jnp
from jax import lax
from jax.experimental import pallas as pl
from jax.experimental.pallas import tpu as pltpu
from jax.experimental.pallas import tpu_sc as plsc

N_NODES = 10000
N_EDGES = 160000
D = 256
HALF = 128
N_LAYERS = 4
N_GRAPHS = 64
EPS = 1e-5

NPAD = 10240
E2 = N_EDGES + N_NODES
N_TILES = 16

ACH = 64
NCH = 168
ETOT = N_TILES * NCH * ACH
NSTRIP = 8
ROWS_PER_TILE = NPAD // N_TILES
ZB = 64
NZB = ROWS_PER_TILE // ZB

PCH = 128
PCHUNKS = ETOT // (32 * PCH)

_MESH = plsc.VectorSubcoreMesh(core_axis_name="c", subcore_axis_name="s")
_SC_PARAMS = pltpu.CompilerParams(needs_layout_passes=False)


@functools.partial(
    pl.kernel,
    out_type=jax.ShapeDtypeStruct((32, NPAD), jnp.float32),
    mesh=_MESH,
    compiler_params=_SC_PARAMS,
    scratch_types=[
        pltpu.VMEM((PCHUNKS, PCH), jnp.int32),
        pltpu.VMEM((PCHUNKS, PCH), jnp.float32),
        pltpu.VMEM((NPAD,), jnp.float32),
    ],
)
def _deg_kernel(dst_hbm, ew_hbm, out_hbm, dstv, eww, degv):
    c = lax.axis_index("c")
    s = lax.axis_index("s")
    w = s * 2 + c

    def zero_body(i, _):
        degv[pl.ds(i * 16, 16)] = jnp.zeros((16,), jnp.float32)
        return 0

    lax.fori_loop(0, NPAD // 16, zero_body, 0)

    pltpu.sync_copy(dst_hbm.at[w], dstv)
    pltpu.sync_copy(ew_hbm.at[w], eww)

    def body(k, _):
        for u in range(PCH // 16):
            d16 = dstv[k, pl.ds(u * 16, 16)]
            e16 = eww[k, pl.ds(u * 16, 16)]
            plsc.addupdate_scatter(degv, [d16], e16)
        return 0

    lax.fori_loop(0, PCHUNKS, body, 0)
    pltpu.sync_copy(degv, out_hbm.at[w])


def _finalize_deg_body(part_ref, dinv_ref):
    deg = jnp.sum(part_ref[...], axis=0, keepdims=True)
    dinv_ref[...] = jnp.where(deg > 0, 1.0 / jnp.sqrt(deg), 0.0)


def _finalize_deg(partials):
    return pl.pallas_call(
        _finalize_deg_body,
        out_shape=jax.ShapeDtypeStruct((1, NPAD), jnp.float32),
    )(partials)


@functools.partial(
    pl.kernel,
    out_type=jax.ShapeDtypeStruct((32, PCHUNKS, PCH), jnp.float32),
    mesh=_MESH,
    compiler_params=_SC_PARAMS,
    scratch_types=[
        pltpu.VMEM((PCHUNKS, PCH), jnp.int32),
        pltpu.VMEM((PCHUNKS, PCH), jnp.int32),
        pltpu.VMEM((PCHUNKS, PCH), jnp.float32),
        pltpu.VMEM((PCHUNKS, PCH), jnp.float32),
        pltpu.VMEM((NPAD,), jnp.float32),
    ],
)
def _norm_kernel(src_hbm, dst_hbm, ew_hbm, dinv_hbm, out_hbm, srcv, dstv, eww, normv, dinvv):
    c = lax.axis_index("c")
    s = lax.axis_index("s")
    w = s * 2 + c

    pltpu.sync_copy(dinv_hbm, dinvv)
    pltpu.sync_copy(src_hbm.at[w], srcv)
    pltpu.sync_copy(dst_hbm.at[w], dstv)
    pltpu.sync_copy(ew_hbm.at[w], eww)

    def body(k, _):
        for u in range(PCH // 16):
            sl = pl.ds(u * 16, 16)
            s16 = srcv[k, sl]
            d16 = dstv[k, sl]
            e16 = eww[k, sl]
            nv = plsc.load_gather(dinvv, [s16]) * e16 * plsc.load_gather(dinvv, [d16])
            normv[k, sl] = nv
        return 0

    lax.fori_loop(0, PCHUNKS, body, 0)
    pltpu.sync_copy(normv, out_hbm.at[w])


@functools.partial(
    pl.kernel,
    out_type=jax.ShapeDtypeStruct((2, NPAD, HALF), jnp.float32),
    mesh=_MESH,
    compiler_params=_SC_PARAMS,
    scratch_types=[
        pltpu.VMEM((NSTRIP, ACH), jnp.int32),
        pltpu.VMEM((NSTRIP, ACH), jnp.int32),
        pltpu.VMEM((NSTRIP, ACH), jnp.float32),
        pltpu.VMEM((ACH, HALF), jnp.float32),
        pltpu.VMEM((ACH, HALF), jnp.float32),
        pltpu.VMEM_SHARED((NPAD, HALF), jnp.float32),
        pltpu.SemaphoreType.DMA,
        pltpu.SemaphoreType.DMA,
        pltpu.SemaphoreType.DMA,
        pltpu.SemaphoreType.DMA,
    ],
)
def _agg_kernel(h_hbm, src_hbm, dst_hbm, norm_hbm, out_hbm,
                srcv, dstv, normv, gb0, gb1, acc, gs0, gs1, ss0, ss1):
    c = lax.axis_index("c")
    s = lax.axis_index("s")
    gbufs = (gb0, gb1)
    gsems = (gs0, gs1)
    ssems = (ss0, ss1)

    def zbody(r, _):
        for u in range(HALF // 16):
            gb0[r, pl.ds(u * 16, 16)] = jnp.zeros((16,), jnp.float32)
        return 0

    lax.fori_loop(0, ACH, zbody, 0)
    for q in range(NZB):
        pltpu.sync_copy(gb0, acc.at[pl.ds(s * ROWS_PER_TILE + q * ZB, ZB)])
    plsc.subcore_barrier()

    def _gather(k2, b):
        pltpu.async_copy(h_hbm.at[c].at[srcv.at[k2]], gbufs[b], gsems[b])

    def _wait_gather(k2, b):
        pltpu.make_async_copy(h_hbm.at[c].at[srcv.at[k2]], gbufs[b], gsems[b]).wait()

    def _scatter(k2, b):
        pltpu.async_copy(gbufs[b], acc.at[dstv.at[k2]], ssems[b], add=True)

    def _wait_scatter(b):
        pltpu.make_async_copy(gbufs[b], acc.at[dstv.at[0]], ssems[b]).wait()

    def strip_body(st, _):
        off = pl.ds(pl.multiple_of(st * NSTRIP, NSTRIP), NSTRIP)
        pltpu.sync_copy(src_hbm.at[s].at[off], srcv)
        pltpu.sync_copy(dst_hbm.at[s].at[off], dstv)
        pltpu.sync_copy(norm_hbm.at[s].at[off], normv)
        _gather(0, 0)
        for k2 in range(NSTRIP):
            b = k2 % 2
            if k2 < NSTRIP - 1:
                if k2 >= 1:
                    _wait_scatter(1 - b)
                _gather(k2 + 1, 1 - b)
            _wait_gather(k2, b)

            idxk = jnp.zeros((16,), jnp.int32) + k2

            def rowbody(r, _, b=b, idxk=idxk):
                sc16 = plsc.load_gather(normv, [idxk, jnp.zeros((16,), jnp.int32) + r])
                for u in range(HALF // 16):
                    sl = pl.ds(u * 16, 16)
                    gbufs[b][r, sl] = gbufs[b][r, sl] * sc16
                return 0

            lax.fori_loop(0, ACH, rowbody, 0)

            _scatter(k2, b)
        _wait_scatter(0)
        _wait_scatter(1)
        return 0

    lax.fori_loop(0, NCH // NSTRIP, strip_body, 0)
    plsc.subcore_barrier()

    for q in range(NZB):
        rows = pl.ds(s * ROWS_PER_TILE + q * ZB, ZB)
        pltpu.sync_copy(acc.at[rows], gb0)
        pltpu.sync_copy(gb0, out_hbm.at[c].at[rows])


_GRID = NPAD // 1024
_NB = 1024


def _matmul_body(x_ref, w_ref, h_ref):
    x0 = x_ref[0]
    x1 = x_ref[1]
    h_ref[0, :, :] = (
        jnp.dot(x0, w_ref[0, 0], preferred_element_type=jnp.float32)
        + jnp.dot(x1, w_ref[1, 0], preferred_element_type=jnp.float32)
    )
    h_ref[1, :, :] = (
        jnp.dot(x0, w_ref[0, 1], preferred_element_type=jnp.float32)
        + jnp.dot(x1, w_ref[1, 1], preferred_element_type=jnp.float32)
    )


def _matmul(x, wq):
    return pl.pallas_call(
        _matmul_body,
        grid=(_GRID,),
        in_specs=[
            pl.BlockSpec((2, _NB, HALF), lambda i: (0, i, 0)),
            pl.BlockSpec((2, 2, HALF, HALF), lambda i: (0, 0, 0, 0)),
        ],
        out_specs=pl.BlockSpec((2, _NB, HALF), lambda i: (0, i, 0)),
        out_shape=jax.ShapeDtypeStruct((2, NPAD, HALF), jnp.float32),
    )(x, wq)


def _gn_fused_body(agg_ref, b_ref, p_ref, gms_ref, gw_ref, gb_ref,
                   x_ref, s1_ref, s2_ref, cnt_ref):
    ph = pl.program_id(0)
    i = pl.program_id(1)
    pb = p_ref[...]
    HI = lax.Precision.HIGHEST

    @pl.when((ph == 0) & (i == 0))
    def _():
        s1_ref[...] = jnp.zeros_like(s1_ref)
        s2_ref[...] = jnp.zeros_like(s2_ref)
        cnt_ref[...] = jnp.zeros_like(cnt_ref)

    @pl.when(ph == 0)
    def _():
        ones = jnp.ones((_NB, HALF), jnp.float32)
        cnt_ref[...] += jnp.dot(pb, ones, preferred_element_type=jnp.float32,
                                precision=HI)
        for f in range(2):
            y = agg_ref[f] + b_ref[f]
            s1_ref[f, :, :] += jnp.dot(pb, y, preferred_element_type=jnp.float32,
                                       precision=HI)

    @pl.when(ph == 1)
    def _():
        cnt = jnp.maximum(cnt_ref[...], 1.0)
        for f in range(2):
            mg = (s1_ref[f] / cnt) * gms_ref[f]
            mb = lax.dot_general(pb, mg, (((0,), (0,)), ((), ())),
                                 preferred_element_type=jnp.float32, precision=HI)
            o = (agg_ref[f] + b_ref[f]) - mb
            s2_ref[f, :, :] += jnp.dot(pb, o * o, preferred_element_type=jnp.float32,
                                       precision=HI)

    @pl.when(ph == 2)
    def _():
        cnt = jnp.maximum(cnt_ref[...], 1.0)
        for f in range(2):
            g = gms_ref[f]
            m = s1_ref[f] / cnt
            rstd = 1.0 / jnp.sqrt(s2_ref[f] / cnt + EPS)
            mb = lax.dot_general(pb, m * g, (((0,), (0,)), ((), ())),
                                 preferred_element_type=jnp.float32, precision=HI)
            rb = lax.dot_general(pb, rstd, (((0,), (0,)), ((), ())),
                                 preferred_element_type=jnp.float32, precision=HI)
            x = ((agg_ref[f] + b_ref[f]) - mb) * rb * gw_ref[f] + gb_ref[f]
            x_ref[f, :, :] = jnp.maximum(x, 0.0)


def _gn_fused(agg, b2, p, gms2, gw2, gb2):
    outs = pl.pallas_call(
        _gn_fused_body,
        grid=(3, _GRID),
        in_specs=[
            pl.BlockSpec((2, _NB, HALF), lambda ph, i: (0, i, 0)),
            pl.BlockSpec((2, 1, HALF), lambda ph, i: (0, 0, 0)),
            pl.BlockSpec((N_GRAPHS, _NB), lambda ph, i: (0, i)),
            pl.BlockSpec((2, 1, HALF), lambda ph, i: (0, 0, 0)),
            pl.BlockSpec((2, 1, HALF), lambda ph, i: (0, 0, 0)),
            pl.BlockSpec((2, 1, HALF), lambda ph, i: (0, 0, 0)),
        ],
        out_specs=[
            pl.BlockSpec((2, _NB, HALF), lambda ph, i: (0, i, 0)),
            pl.BlockSpec((2, N_GRAPHS, HALF), lambda ph, i: (0, 0, 0)),
            pl.BlockSpec((2, N_GRAPHS, HALF), lambda ph, i: (0, 0, 0)),
            pl.BlockSpec((N_GRAPHS, HALF), lambda ph, i: (0, 0)),
        ],
        out_shape=[
            jax.ShapeDtypeStruct((2, NPAD, HALF), jnp.float32),
            jax.ShapeDtypeStruct((2, N_GRAPHS, HALF), jnp.float32),
            jax.ShapeDtypeStruct((2, N_GRAPHS, HALF), jnp.float32),
            jax.ShapeDtypeStruct((N_GRAPHS, HALF), jnp.float32),
        ],
    )(agg, b2, p, gms2, gw2, gb2)
    return outs[0]


def kernel(node, edge_index, edge_attr, batch_ptr, W, b, gn_weight, gn_bias, gn_mean_scale):
    loop = jnp.arange(N_NODES, dtype=jnp.int32)
    src2 = jnp.concatenate([edge_index[0], loop])
    dst2 = jnp.concatenate([edge_index[1], loop])
    ew2 = jnp.concatenate([edge_attr, jnp.ones((N_NODES,), jnp.float32)])
    pad = ETOT - E2
    src_flat = jnp.pad(src2, (0, pad))
    dst_flat = jnp.pad(dst2, (0, pad))
    ew32 = jnp.pad(ew2, (0, pad)).reshape(32, PCHUNKS, PCH)
    src32 = src_flat.reshape(32, PCHUNKS, PCH)
    dst32 = dst_flat.reshape(32, PCHUNKS, PCH)
    src_slab = src_flat.reshape(N_TILES, NCH, ACH)
    dst_slab = dst_flat.reshape(N_TILES, NCH, ACH)

    p = (batch_ptr[None, :] == jnp.arange(N_GRAPHS, dtype=jnp.int32)[:, None])
    p = jnp.pad(p.astype(jnp.float32), ((0, 0), (0, NPAD - N_NODES)))

    xpad = jnp.pad(node, ((0, NPAD - N_NODES), (0, 0)))
    x = jnp.stack([xpad[:, :HALF], xpad[:, HALF:]])

    partials = _deg_kernel(dst32, ew32)
    dinv = _finalize_deg(partials).reshape(NPAD)
    norm_slab = _norm_kernel(src32, dst32, ew32, dinv).reshape(N_TILES, NCH, ACH)

    for l in range(N_LAYERS):
        wq = W[l].reshape(2, HALF, 2, HALF).swapaxes(1, 2)
        b2 = b[l].reshape(2, 1, HALF)
        gms2 = gn_mean_scale[l].reshape(2, 1, HALF)
        gw2 = gn_weight[l].reshape(2, 1, HALF)
        gb2 = gn_bias[l].reshape(2, 1, HALF)

        h = _matmul(x, wq)
        agg = _agg_kernel(h, src_slab, dst_slab, norm_slab)
        x = _gn_fused(agg, b2, p, gms2, gw2, gb2)

    return jnp.concatenate([x[0, :N_NODES, :], x[1, :N_NODES, :]], axis=1)


# speedup vs baseline: 1.0890x; 1.0829x over previous
import functools

import jax
import jax.numpy as jnp
from jax import lax
from jax.experimental import pallas as pl
from jax.experimental.pallas import tpu as pltpu
from jax.experimental.pallas import tpu_sc as plsc

N_NODES = 10000
N_EDGES = 160000
D = 256
HALF = 128
N_LAYERS = 4
N_GRAPHS = 64
EPS = 1e-5

NPAD = 10240
E2 = N_EDGES + N_NODES
N_TILES = 16

ACH = 64
NCH = 168
ETOT = N_TILES * NCH * ACH
NSTRIP = 8
ROWS_PER_TILE = NPAD // N_TILES
ZB = 64
NZB = ROWS_PER_TILE // ZB

PCH = 128
PCHUNKS = ETOT // (32 * PCH)

_MESH = plsc.VectorSubcoreMesh(core_axis_name="c", subcore_axis_name="s")
_SC_PARAMS = pltpu.CompilerParams(needs_layout_passes=False)


@functools.partial(
    pl.kernel,
    out_type=jax.ShapeDtypeStruct((32, NPAD), jnp.float32),
    mesh=_MESH,
    compiler_params=_SC_PARAMS,
    scratch_types=[
        pltpu.VMEM((PCHUNKS, PCH), jnp.int32),
        pltpu.VMEM((PCHUNKS, PCH), jnp.float32),
        pltpu.VMEM((NPAD,), jnp.float32),
    ],
)
def _deg_kernel(dst_hbm, ew_hbm, out_hbm, dstv, eww, degv):
    c = lax.axis_index("c")
    s = lax.axis_index("s")
    w = s * 2 + c

    def zero_body(i, _):
        degv[pl.ds(i * 16, 16)] = jnp.zeros((16,), jnp.float32)
        return 0

    lax.fori_loop(0, NPAD // 16, zero_body, 0)

    pltpu.sync_copy(dst_hbm.at[w], dstv)
    pltpu.sync_copy(ew_hbm.at[w], eww)

    def body(k, _):
        for u in range(PCH // 16):
            d16 = dstv[k, pl.ds(u * 16, 16)]
            e16 = eww[k, pl.ds(u * 16, 16)]
            plsc.addupdate_scatter(degv, [d16], e16)
        return 0

    lax.fori_loop(0, PCHUNKS, body, 0)
    pltpu.sync_copy(degv, out_hbm.at[w])


def _finalize_deg_body(part_ref, dinv_ref):
    deg = jnp.sum(part_ref[...], axis=0, keepdims=True)
    dinv_ref[...] = jnp.where(deg > 0, 1.0 / jnp.sqrt(deg), 0.0)


def _finalize_deg(partials):
    return pl.pallas_call(
        _finalize_deg_body,
        out_shape=jax.ShapeDtypeStruct((1, NPAD), jnp.float32),
    )(partials)


@functools.partial(
    pl.kernel,
    out_type=jax.ShapeDtypeStruct((32, PCHUNKS, PCH), jnp.float32),
    mesh=_MESH,
    compiler_params=_SC_PARAMS,
    scratch_types=[
        pltpu.VMEM((PCHUNKS, PCH), jnp.int32),
        pltpu.VMEM((PCHUNKS, PCH), jnp.int32),
        pltpu.VMEM((PCHUNKS, PCH), jnp.float32),
        pltpu.VMEM((PCHUNKS, PCH), jnp.float32),
        pltpu.VMEM((NPAD,), jnp.float32),
    ],
)
def _norm_kernel(src_hbm, dst_hbm, ew_hbm, dinv_hbm, out_hbm, srcv, dstv, eww, normv, dinvv):
    c = lax.axis_index("c")
    s = lax.axis_index("s")
    w = s * 2 + c

    pltpu.sync_copy(dinv_hbm, dinvv)
    pltpu.sync_copy(src_hbm.at[w], srcv)
    pltpu.sync_copy(dst_hbm.at[w], dstv)
    pltpu.sync_copy(ew_hbm.at[w], eww)

    def body(k, _):
        for u in range(PCH // 16):
            sl = pl.ds(u * 16, 16)
            s16 = srcv[k, sl]
            d16 = dstv[k, sl]
            e16 = eww[k, sl]
            nv = plsc.load_gather(dinvv, [s16]) * e16 * plsc.load_gather(dinvv, [d16])
            normv[k, sl] = nv
        return 0

    lax.fori_loop(0, PCHUNKS, body, 0)
    pltpu.sync_copy(normv, out_hbm.at[w])


@functools.partial(
    pl.kernel,
    out_type=jax.ShapeDtypeStruct((2, NPAD, HALF), jnp.float32),
    mesh=_MESH,
    compiler_params=_SC_PARAMS,
    scratch_types=[
        pltpu.VMEM((NSTRIP, ACH), jnp.int32),
        pltpu.VMEM((NSTRIP, ACH), jnp.int32),
        pltpu.VMEM((NSTRIP, ACH), jnp.float32),
        pltpu.VMEM((ACH, HALF), jnp.float32),
        pltpu.VMEM((ACH, HALF), jnp.float32),
        pltpu.VMEM((ACH, HALF), jnp.float32),
        pltpu.VMEM_SHARED((NPAD, HALF), jnp.float32),
        pltpu.SemaphoreType.DMA,
        pltpu.SemaphoreType.DMA,
        pltpu.SemaphoreType.DMA,
        pltpu.SemaphoreType.DMA,
        pltpu.SemaphoreType.DMA,
        pltpu.SemaphoreType.DMA,
    ],
)
def _agg_kernel(h_hbm, src_hbm, dst_hbm, norm_hbm, out_hbm,
                srcv, dstv, normv, gb0, gb1, gb2, acc,
                gs0, gs1, gs2, ss0, ss1, ss2):
    c = lax.axis_index("c")
    s = lax.axis_index("s")
    gbufs = (gb0, gb1, gb2)
    gsems = (gs0, gs1, gs2)
    ssems = (ss0, ss1, ss2)

    def zbody(r, _):
        for u in range(HALF // 16):
            gb0[r, pl.ds(u * 16, 16)] = jnp.zeros((16,), jnp.float32)
        return 0

    lax.fori_loop(0, ACH, zbody, 0)
    for q in range(NZB):
        pltpu.sync_copy(gb0, acc.at[pl.ds(s * ROWS_PER_TILE + q * ZB, ZB)])
    plsc.subcore_barrier()

    def _gather(k2, b):
        pltpu.async_copy(h_hbm.at[c].at[srcv.at[k2]], gbufs[b], gsems[b])

    def _wait_gather(k2, b):
        pltpu.make_async_copy(h_hbm.at[c].at[srcv.at[k2]], gbufs[b], gsems[b]).wait()

    def _scatter(k2, b):
        pltpu.async_copy(gbufs[b], acc.at[dstv.at[k2]], ssems[b], add=True)

    def _wait_scatter(b):
        pltpu.make_async_copy(gbufs[b], acc.at[dstv.at[0]], ssems[b]).wait()

    def strip_body(st, _):
        off = pl.ds(pl.multiple_of(st * NSTRIP, NSTRIP), NSTRIP)
        pltpu.sync_copy(src_hbm.at[s].at[off], srcv)
        pltpu.sync_copy(dst_hbm.at[s].at[off], dstv)
        pltpu.sync_copy(norm_hbm.at[s].at[off], normv)
        _gather(0, 0)
        for k2 in range(NSTRIP):
            b = k2 % 3
            if k2 < NSTRIP - 1:
                nb = (k2 + 1) % 3
                if k2 >= 2:
                    _wait_scatter(nb)
                _gather(k2 + 1, nb)
            _wait_gather(k2, b)

            idxk = jnp.zeros((16,), jnp.int32) + k2

            def rowbody(r, _, b=b, idxk=idxk):
                sc16 = plsc.load_gather(normv, [idxk, jnp.zeros((16,), jnp.int32) + r])
                for u in range(HALF // 16):
                    sl = pl.ds(u * 16, 16)
                    gbufs[b][r, sl] = gbufs[b][r, sl] * sc16
                return 0

            lax.fori_loop(0, ACH, rowbody, 0)

            _scatter(k2, b)
        for b in range(3):
            _wait_scatter(b)
        return 0

    lax.fori_loop(0, NCH // NSTRIP, strip_body, 0)
    plsc.subcore_barrier()

    for q in range(NZB):
        rows = pl.ds(s * ROWS_PER_TILE + q * ZB, ZB)
        pltpu.sync_copy(acc.at[rows], gb0)
        pltpu.sync_copy(gb0, out_hbm.at[c].at[rows])


_GRID = NPAD // 1024
_NB = 1024


def _matmul_body(x_ref, w_ref, h_ref):
    x0 = x_ref[0]
    x1 = x_ref[1]
    h_ref[0, :, :] = (
        jnp.dot(x0, w_ref[0, 0], preferred_element_type=jnp.float32)
        + jnp.dot(x1, w_ref[1, 0], preferred_element_type=jnp.float32)
    )
    h_ref[1, :, :] = (
        jnp.dot(x0, w_ref[0, 1], preferred_element_type=jnp.float32)
        + jnp.dot(x1, w_ref[1, 1], preferred_element_type=jnp.float32)
    )


def _matmul(x, wq):
    return pl.pallas_call(
        _matmul_body,
        grid=(_GRID,),
        in_specs=[
            pl.BlockSpec((2, _NB, HALF), lambda i: (0, i, 0)),
            pl.BlockSpec((2, 2, HALF, HALF), lambda i: (0, 0, 0, 0)),
        ],
        out_specs=pl.BlockSpec((2, _NB, HALF), lambda i: (0, i, 0)),
        out_shape=jax.ShapeDtypeStruct((2, NPAD, HALF), jnp.float32),
    )(x, wq)


def _gn_stats_body(agg_ref, b_ref, p_ref, y_ref, s1_ref, cnt_ref):
    i = pl.program_id(0)

    @pl.when(i == 0)
    def _():
        s1_ref[...] = jnp.zeros_like(s1_ref)
        cnt_ref[...] = jnp.zeros_like(cnt_ref)

    pb = p_ref[...]
    ones = jnp.ones((_NB, HALF), jnp.float32)
    cnt_ref[...] += jnp.dot(pb, ones, preferred_element_type=jnp.float32, precision=lax.Precision.HIGHEST)
    for f in range(2):
        y = agg_ref[f] + b_ref[f]
        y_ref[f, :, :] = y
        s1_ref[f, :, :] += jnp.dot(pb, y, preferred_element_type=jnp.float32, precision=lax.Precision.HIGHEST)


def _gn_stats(agg, b2, p):
    return pl.pallas_call(
        _gn_stats_body,
        grid=(_GRID,),
        in_specs=[
            pl.BlockSpec((2, _NB, HALF), lambda i: (0, i, 0)),
            pl.BlockSpec((2, 1, HALF), lambda i: (0, 0, 0)),
            pl.BlockSpec((N_GRAPHS, _NB), lambda i: (0, i)),
        ],
        out_specs=[
            pl.BlockSpec((2, _NB, HALF), lambda i: (0, i, 0)),
            pl.BlockSpec((2, N_GRAPHS, HALF), lambda i: (0, 0, 0)),
            pl.BlockSpec((N_GRAPHS, HALF), lambda i: (0, 0)),
        ],
        out_shape=[
            jax.ShapeDtypeStruct((2, NPAD, HALF), jnp.float32),
            jax.ShapeDtypeStruct((2, N_GRAPHS, HALF), jnp.float32),
            jax.ShapeDtypeStruct((N_GRAPHS, HALF), jnp.float32),
        ],
    )(agg, b2, p)


def _gn_var_body(y_ref, p_ref, s1_ref, cnt_ref, gms_ref, s2_ref):
    i = pl.program_id(0)

    @pl.when(i == 0)
    def _():
        s2_ref[...] = jnp.zeros_like(s2_ref)

    pb = p_ref[...]
    cnt = jnp.maximum(cnt_ref[...], 1.0)
    for f in range(2):
        mg = (s1_ref[f] / cnt) * gms_ref[f]
        mb = lax.dot_general(pb, mg, (((0,), (0,)), ((), ())),
                             preferred_element_type=jnp.float32, precision=lax.Precision.HIGHEST)
        o = y_ref[f] - mb
        s2_ref[f, :, :] += jnp.dot(pb, o * o, preferred_element_type=jnp.float32, precision=lax.Precision.HIGHEST)


def _gn_var(y, p, s1, cnt, gms2):
    return pl.pallas_call(
        _gn_var_body,
        grid=(_GRID,),
        in_specs=[
            pl.BlockSpec((2, _NB, HALF), lambda i: (0, i, 0)),
            pl.BlockSpec((N_GRAPHS, _NB), lambda i: (0, i)),
            pl.BlockSpec((2, N_GRAPHS, HALF), lambda i: (0, 0, 0)),
            pl.BlockSpec((N_GRAPHS, HALF), lambda i: (0, 0)),
            pl.BlockSpec((2, 1, HALF), lambda i: (0, 0, 0)),
        ],
        out_specs=pl.BlockSpec((2, N_GRAPHS, HALF), lambda i: (0, 0, 0)),
        out_shape=jax.ShapeDtypeStruct((2, N_GRAPHS, HALF), jnp.float32),
    )(y, p, s1, cnt, gms2)


def _gn_apply_body(y_ref, p_ref, s1_ref, s2_ref, cnt_ref, gms_ref, gw_ref, gb_ref, x_ref):
    pb = p_ref[...]
    cnt = jnp.maximum(cnt_ref[...], 1.0)
    for f in range(2):
        g = gms_ref[f]
        m = s1_ref[f] / cnt
        var = s2_ref[f] / cnt
        rstd = 1.0 / jnp.sqrt(var + EPS)
        mg = m * g
        mb = lax.dot_general(pb, mg, (((0,), (0,)), ((), ())),
                             preferred_element_type=jnp.float32, precision=lax.Precision.HIGHEST)
        rb = lax.dot_general(pb, rstd, (((0,), (0,)), ((), ())),
                             preferred_element_type=jnp.float32, precision=lax.Precision.HIGHEST)
        x = (y_ref[f] - mb) * rb * gw_ref[f] + gb_ref[f]
        x_ref[f, :, :] = jnp.maximum(x, 0.0)


def _gn_apply(y, p, s1, s2, cnt, gms2, gw2, gb2):
    return pl.pallas_call(
        _gn_apply_body,
        grid=(_GRID,),
        in_specs=[
            pl.BlockSpec((2, _NB, HALF), lambda i: (0, i, 0)),
            pl.BlockSpec((N_GRAPHS, _NB), lambda i: (0, i)),
            pl.BlockSpec((2, N_GRAPHS, HALF), lambda i: (0, 0, 0)),
            pl.BlockSpec((2, N_GRAPHS, HALF), lambda i: (0, 0, 0)),
            pl.BlockSpec((N_GRAPHS, HALF), lambda i: (0, 0)),
            pl.BlockSpec((2, 1, HALF), lambda i: (0, 0, 0)),
            pl.BlockSpec((2, 1, HALF), lambda i: (0, 0, 0)),
            pl.BlockSpec((2, 1, HALF), lambda i: (0, 0, 0)),
        ],
        out_specs=pl.BlockSpec((2, _NB, HALF), lambda i: (0, i, 0)),
        out_shape=jax.ShapeDtypeStruct((2, NPAD, HALF), jnp.float32),
    )(y, p, s1, s2, cnt, gms2, gw2, gb2)


def kernel(node, edge_index, edge_attr, batch_ptr, W, b, gn_weight, gn_bias, gn_mean_scale):
    loop = jnp.arange(N_NODES, dtype=jnp.int32)
    src2 = jnp.concatenate([edge_index[0], loop])
    dst2 = jnp.concatenate([edge_index[1], loop])
    ew2 = jnp.concatenate([edge_attr, jnp.ones((N_NODES,), jnp.float32)])
    pad = ETOT - E2
    src_flat = jnp.pad(src2, (0, pad))
    dst_flat = jnp.pad(dst2, (0, pad))
    ew32 = jnp.pad(ew2, (0, pad)).reshape(32, PCHUNKS, PCH)
    src32 = src_flat.reshape(32, PCHUNKS, PCH)
    dst32 = dst_flat.reshape(32, PCHUNKS, PCH)
    src_slab = src_flat.reshape(N_TILES, NCH, ACH)
    dst_slab = dst_flat.reshape(N_TILES, NCH, ACH)

    p = (batch_ptr[None, :] == jnp.arange(N_GRAPHS, dtype=jnp.int32)[:, None])
    p = jnp.pad(p.astype(jnp.float32), ((0, 0), (0, NPAD - N_NODES)))

    xpad = jnp.pad(node, ((0, NPAD - N_NODES), (0, 0)))
    x = jnp.stack([xpad[:, :HALF], xpad[:, HALF:]])

    partials = _deg_kernel(dst32, ew32)
    dinv = _finalize_deg(partials).reshape(NPAD)
    norm_slab = _norm_kernel(src32, dst32, ew32, dinv).reshape(N_TILES, NCH, ACH)

    for l in range(N_LAYERS):
        wq = W[l].reshape(2, HALF, 2, HALF).swapaxes(1, 2)
        b2 = b[l].reshape(2, 1, HALF)
        gms2 = gn_mean_scale[l].reshape(2, 1, HALF)
        gw2 = gn_weight[l].reshape(2, 1, HALF)
        gb2 = gn_bias[l].reshape(2, 1, HALF)

        h = _matmul(x, wq)
        agg = _agg_kernel(h, src_slab, dst_slab, norm_slab)
        y, s1, cnt = _gn_stats(agg, b2, p)
        s2 = _gn_var(y, p, s1, cnt, gms2)
        x = _gn_apply(y, p, s1, s2, cnt, gms2, gw2, gb2)

    return jnp.concatenate([x[0, :N_NODES, :], x[1, :N_NODES, :]], axis=1)
